# speedup vs baseline: 14.2226x; 14.2226x over previous
import jax
import jax.numpy as jnp
from jax.experimental import pallas as pl
from jax.experimental.pallas import tpu as pltpu

N = 1000
K = 32
NP = 1024
DIN = 2005
DINP = 2048
MLPH = 512
HID = 128
OUT = 128
NEG = -1.0e30


def _adj_body(x_ref, w1_ref, b1_ref, w2_ref, b2_ref, adj_ref):
    h = jnp.dot(x_ref[...], w1_ref[...], preferred_element_type=jnp.float32)
    h = jnp.maximum(h + b1_ref[...], 0.0)
    adj_ref[...] = (
        jnp.dot(h, w2_ref[...], preferred_element_type=jnp.float32) + b2_ref[...]
    )


def _cumsum_lanes(x):
    rows, n = x.shape
    s = 1
    while s < n:
        shifted = jnp.concatenate(
            [jnp.zeros((rows, s), x.dtype), x[:, : n - s]], axis=1
        )
        x = x + shifted
        s *= 2
    return x


def _select_body(adj_ref, a_ref):
    adj = adj_ref[...]
    rows = adj.shape[0]
    b = jax.lax.bitcast_convert_type(adj, jnp.int32)
    key = jnp.where(b >= 0, b, b ^ jnp.int32(0x7FFFFFFF))

    def step(i, p):
        c = p + (jnp.int32(1) << (jnp.int32(31) - i))
        cnt = jnp.sum((key >= c).astype(jnp.int32), axis=1, keepdims=True)
        return jnp.where(cnt >= K, c, p)

    p0 = jnp.full((rows, 1), jnp.int32(-2147483648))
    t = jax.lax.fori_loop(0, 32, step, p0)

    gt = key > t
    eq = key == t
    cgt = jnp.sum(gt.astype(jnp.int32), axis=1, keepdims=True)
    need = K - cgt
    eqcs = _cumsum_lanes(eq.astype(jnp.int32))
    mask = gt | (eq & (eqcs <= need))
    a_ref[...] = jnp.where(mask, adj, 0.0)


def _gcn_body(
    a_ref, ne_ref, wc1_ref, bc1_ref, wc2_ref, bc2_ref, gamma_ref, beta_ref, out_ref
):
    A = a_ref[...]
    deg = 1.0 + jnp.sum(A, axis=1, keepdims=True)
    dinv = jnp.where(deg > 0, jax.lax.rsqrt(deg), 0.0)

    h1 = jnp.dot(ne_ref[...], wc1_ref[...], preferred_element_type=jnp.float32)
    agg1 = (
        dinv * jnp.dot(A, dinv * h1, preferred_element_type=jnp.float32)
        + (dinv * dinv) * h1
        + bc1_ref[...]
    )

    rm = (jax.lax.broadcasted_iota(jnp.int32, (NP, 1), 0) < N).astype(jnp.float32)
    mean = jnp.sum(agg1 * rm, axis=0, keepdims=True) / N
    var = jnp.sum(((agg1 - mean) ** 2) * rm, axis=0, keepdims=True) / N
    o1 = gamma_ref[...] * (agg1 - mean) * jax.lax.rsqrt(var + 1e-5) + beta_ref[...]
    o1 = jnp.maximum(o1, 0.0)

    h2 = jnp.dot(o1, wc2_ref[...], preferred_element_type=jnp.float32)
    out_ref[...] = (
        dinv * jnp.dot(A, dinv * h2, preferred_element_type=jnp.float32)
        + (dinv * dinv) * h2
        + bc2_ref[...]
    )


def kernel(probs, bbox_coords, query_embeddings, node_embeddings,
           W1, b1, W2, b2, Wc1, bc1, Wc2, bc2, gamma, beta):
    f32 = jnp.float32
    ew = jnp.concatenate([query_embeddings, probs, bbox_coords], axis=1)
    X = jnp.pad(ew, ((0, NP - N), (0, DINP - DIN)))
    W1p = jnp.pad(W1, ((0, DINP - DIN), (0, 0)))
    W2p = jnp.pad(W2, ((0, 0), (0, NP - N)))
    b2p = jnp.pad(b2, (0, NP - N), constant_values=NEG)
    nep = jnp.pad(node_embeddings, ((0, NP - N), (0, NP - N)))
    Wc1p = jnp.pad(Wc1, ((0, NP - N), (0, 0)))

    adj = pl.pallas_call(
        _adj_body,
        out_shape=jax.ShapeDtypeStruct((NP, NP), f32),
    )(X, W1p, b1.reshape(1, MLPH), W2p, b2p.reshape(1, NP))

    A = pl.pallas_call(
        _select_body,
        out_shape=jax.ShapeDtypeStruct((NP, NP), f32),
    )(adj)

    out = pl.pallas_call(
        _gcn_body,
        out_shape=jax.ShapeDtypeStruct((NP, OUT), f32),
    )(
        A, nep, Wc1p,
        bc1.reshape(1, HID), Wc2, bc2.reshape(1, OUT),
        gamma.reshape(1, HID), beta.reshape(1, HID),
    )
    return out[:N]


# speedup vs baseline: 18.7680x; 1.3196x over previous
import jax
import jax.numpy as jnp
from jax.experimental import pallas as pl
from jax.experimental.pallas import tpu as pltpu

N = 1000
K = 32
RB = 200
P = 1001
MLPH = 512
HID = 128
OUT = 128


def _cumsum_lanes(x):
    rows, n = x.shape
    s = 1
    while s < n:
        shifted = jnp.concatenate(
            [jnp.zeros((rows, s), x.dtype), x[:, : n - s]], axis=1
        )
        x = x + shifted
        s *= 2
    return x


def _topk_mask(adj):
    rows = adj.shape[0]
    b = jax.lax.bitcast_convert_type(adj, jnp.int32)
    key = jnp.where(b >= 0, b, b ^ jnp.int32(0x7FFFFFFF))

    def step(i, p):
        c = p + (jnp.int32(1) << (jnp.int32(31) - i))
        cnt = jnp.sum((key >= c).astype(jnp.int32), axis=1, keepdims=True)
        return jnp.where(cnt >= K, c, p)

    p0 = jnp.full((rows, 1), jnp.int32(-2147483648))
    t = jax.lax.fori_loop(0, 32, step, p0)

    gt = key > t
    eq = key == t
    cgt = jnp.sum(gt.astype(jnp.int32), axis=1, keepdims=True)
    need = K - cgt
    eqcs = _cumsum_lanes(eq.astype(jnp.int32))
    mask = gt | (eq & (eqcs <= need))
    return jnp.where(mask, adj, 0.0)


def _adj_sel_body(q_ref, p_ref, bb_ref, w1_ref, w1c_ref, b1_ref, w2_ref,
                  b2_ref, a_ref):
    h = jnp.dot(q_ref[...], w1_ref[0:N, :], preferred_element_type=jnp.float32)
    h = h + jnp.dot(p_ref[...], w1_ref[N:N + P, :],
                    preferred_element_type=jnp.float32)
    h = h + jnp.dot(bb_ref[...], w1c_ref[...],
                    preferred_element_type=jnp.float32)
    h = jnp.maximum(h + b1_ref[...], 0.0)
    adj = jnp.dot(h, w2_ref[...], preferred_element_type=jnp.float32) + b2_ref[...]
    a_ref[...] = _topk_mask(adj)


def _gcn_body(a_ref, ne_ref, wc1_ref, bc1_ref, wc2_ref, bc2_ref,
              gamma_ref, beta_ref, out_ref):
    A = a_ref[...]
    deg = 1.0 + jnp.sum(A, axis=1, keepdims=True)
    dinv = jnp.where(deg > 0, jax.lax.rsqrt(deg), 0.0)

    h1 = jnp.dot(ne_ref[...], wc1_ref[...], preferred_element_type=jnp.float32)
    agg1 = (
        dinv * jnp.dot(A, dinv * h1, preferred_element_type=jnp.float32)
        + (dinv * dinv) * h1
        + bc1_ref[...]
    )

    mean = jnp.sum(agg1, axis=0, keepdims=True) / N
    var = jnp.sum((agg1 - mean) ** 2, axis=0, keepdims=True) / N
    o1 = gamma_ref[...] * (agg1 - mean) * jax.lax.rsqrt(var + 1e-5) + beta_ref[...]
    o1 = jnp.maximum(o1, 0.0)

    h2 = jnp.dot(o1, wc2_ref[...], preferred_element_type=jnp.float32)
    out_ref[...] = (
        dinv * jnp.dot(A, dinv * h2, preferred_element_type=jnp.float32)
        + (dinv * dinv) * h2
        + bc2_ref[...]
    )


def kernel(probs, bbox_coords, query_embeddings, node_embeddings,
           W1, b1, W2, b2, Wc1, bc1, Wc2, bc2, gamma, beta):
    f32 = jnp.float32
    W1c = W1[N + P:, :]

    nblk = N // RB
    A = pl.pallas_call(
        _adj_sel_body,
        grid=(nblk,),
        in_specs=[
            pl.BlockSpec((RB, N), lambda i: (i, 0)),
            pl.BlockSpec((RB, P), lambda i: (i, 0)),
            pl.BlockSpec((RB, 4), lambda i: (i, 0)),
            pl.BlockSpec((N + P + 4, MLPH), lambda i: (0, 0)),
            pl.BlockSpec((4, MLPH), lambda i: (0, 0)),
            pl.BlockSpec((1, MLPH), lambda i: (0, 0)),
            pl.BlockSpec((MLPH, N), lambda i: (0, 0)),
            pl.BlockSpec((1, N), lambda i: (0, 0)),
        ],
        out_specs=pl.BlockSpec((RB, N), lambda i: (i, 0)),
        out_shape=jax.ShapeDtypeStruct((N, N), f32),
        compiler_params=pltpu.CompilerParams(
            dimension_semantics=("arbitrary",),
        ),
    )(query_embeddings, probs, bbox_coords, W1, W1c,
      b1.reshape(1, MLPH), W2, b2.reshape(1, N))

    out = pl.pallas_call(
        _gcn_body,
        out_shape=jax.ShapeDtypeStruct((N, OUT), f32),
    )(
        A, node_embeddings, Wc1,
        bc1.reshape(1, HID), Wc2, bc2.reshape(1, OUT),
        gamma.reshape(1, HID), beta.reshape(1, HID),
    )
    return out
